# baseline (speedup 1.0000x reference)
import functools

import jax
import jax.numpy as jnp
from jax import lax
from jax.experimental import pallas as pl
from jax.experimental.pallas import tpu as pltpu
from jax.experimental.pallas import tpu_sc as plsc

D = 128
G = 64
MAX_DIS = 5
RB = 256
NW = 32
CHUNK = 128
NSUB = 16


NBUF = 2
PIECE = 40
C0_FRAC = 0.25


def _make_sc_agg(npad, total_chunks, c0):
    rows_per_tile = npad // NSUB
    c1 = total_chunks // NSUB - c0
    assert c0 % PIECE == 0 and c1 % PIECE == 0 and PIECE % NBUF == 0
    mesh = plsc.VectorSubcoreMesh(core_axis_name="c", subcore_axis_name="s")

    @functools.partial(
        pl.kernel,
        mesh=mesh,
        out_type=jax.ShapeDtypeStruct((2, npad, D), jnp.float32),
        scratch_types=[
            pltpu.VMEM((PIECE, CHUNK), jnp.int32),
            pltpu.VMEM((PIECE, CHUNK), jnp.int32),
            pltpu.VMEM((CHUNK, D), jnp.float32),
            pltpu.VMEM((CHUNK, D), jnp.float32),
            pltpu.VMEM_SHARED((npad, D), jnp.float32),
            pltpu.SemaphoreType.DMA,
            pltpu.SemaphoreType.DMA,
        ],
    )
    def sc_agg(m_hbm, src_hbm, dst_hbm, zeros_hbm, out_hbm,
               src_v, dst_v, b0, b1, agg, g0, g1):
        bufs = [b0, b1]
        gsems = [g0, g1]
        cid = lax.axis_index("c")
        sid = lax.axis_index("s")
        base = sid * rows_per_tile
        pltpu.sync_copy(zeros_hbm, agg.at[pl.ds(base, rows_per_tile)])
        plsc.subcore_barrier()

        mycount = c0 + cid * (c1 - c0)
        cbase = cid * NSUB * c0 + sid * mycount

        def piece_body(p, carry):
            off = cbase + p * PIECE
            pltpu.sync_copy(src_hbm.at[pl.ds(off, PIECE)], src_v)
            pltpu.sync_copy(dst_hbm.at[pl.ds(off, PIECE)], dst_v)

            for b in range(NBUF):
                pltpu.async_copy(m_hbm.at[src_v.at[b]], bufs[b], gsems[b])

            def body(i, carry2):
                j0 = i * NBUF
                for b in range(NBUF):
                    jn = j0 + b + NBUF
                    pltpu.make_async_copy(
                        m_hbm.at[src_v.at[j0 + b]], bufs[b], gsems[b]).wait()
                    pltpu.sync_copy(bufs[b], agg.at[dst_v.at[j0 + b]],
                                    add=True)

                    @pl.when(jn < PIECE)
                    def _():
                        pltpu.async_copy(m_hbm.at[src_v.at[jn]], bufs[b],
                                         gsems[b])
                return carry2

            lax.fori_loop(0, PIECE // NBUF, body, 0)
            return carry

        lax.fori_loop(0, mycount // PIECE, piece_body, 0)

        plsc.subcore_barrier()
        pltpu.sync_copy(agg.at[pl.ds(base, rows_per_tile)],
                        out_hbm.at[cid, pl.ds(base, rows_per_tile)])

    return sc_agg


def _embed_body(d_ref, b_ref, emb_ref, wl_ref, bl_ref,
                x_ref, m_ref, cnt_ref):
    i = pl.program_id(0)
    dd = jnp.clip(d_ref[0, 0, :], 0, MAX_DIS)
    oh = (lax.broadcasted_iota(jnp.int32, (RB, 8), 1) == dd[:, None])
    x = jnp.dot(oh.astype(jnp.float32), emb_ref[...],
                preferred_element_type=jnp.float32, precision=lax.Precision.HIGHEST)
    x_ref[...] = x
    m_ref[...] = jnp.maximum(
        jnp.dot(x, wl_ref[...], preferred_element_type=jnp.float32)
        + bl_ref[...], 0.0)
    bb = b_ref[0, 0, :]
    bsel = (lax.broadcasted_iota(jnp.int32, (G, RB), 0) == bb[None, :])
    cnt = jnp.dot(bsel.astype(jnp.float32), jnp.ones((RB, D), jnp.float32),
                  preferred_element_type=jnp.float32, precision=lax.Precision.HIGHEST)

    @pl.when(i == 0)
    def _():
        cnt_ref[...] = cnt

    @pl.when(i > 0)
    def _():
        cnt_ref[...] += cnt


def _gin_update(x_ref, a0_ref, a1_ref, eps_ref, wi_ref, bi_ref,
                wo_ref, bo_ref):
    h = x_ref[...] * (1.0 + eps_ref[0, 0]) + a0_ref[...] + a1_ref[...]
    t = jnp.maximum(
        jnp.dot(h, wi_ref[...], preferred_element_type=jnp.float32)
        + bi_ref[...], 0.0)
    h2 = jnp.dot(t, wo_ref[...], preferred_element_type=jnp.float32) \
        + bo_ref[...]
    return jnp.maximum(h2, 0.0)


def _pool_accum(i, nblk, xn, b_ref, cnt_ref, hm_ref):
    bb = b_ref[0, 0, :]
    bsel = (lax.broadcasted_iota(jnp.int32, (G, RB), 0) == bb[None, :])
    contrib = jnp.dot(bsel.astype(jnp.float32), xn,
                      preferred_element_type=jnp.float32, precision=lax.Precision.HIGHEST)

    @pl.when(i == 0)
    def _():
        hm_ref[...] = contrib

    @pl.when(i > 0)
    def _():
        hm_ref[...] += contrib

    @pl.when(i == nblk - 1)
    def _():
        hm_ref[...] = hm_ref[...] / jnp.maximum(cnt_ref[...], 1.0)


def _mid_body(nblk, x_ref, a0_ref, a1_ref, b_ref, cnt_ref, eps_ref,
              wi_ref, bi_ref, wo_ref, bo_ref, wn_ref, bn_ref,
              xn_ref, mn_ref, hm_ref):
    i = pl.program_id(0)
    xn = _gin_update(x_ref, a0_ref, a1_ref, eps_ref, wi_ref, bi_ref,
                     wo_ref, bo_ref)
    xn_ref[...] = xn
    mn_ref[...] = jnp.maximum(
        jnp.dot(xn, wn_ref[...], preferred_element_type=jnp.float32)
        + bn_ref[...], 0.0)
    _pool_accum(i, nblk, xn, b_ref, cnt_ref, hm_ref)


def _last_body(nblk, x_ref, a0_ref, a1_ref, b_ref, cnt_ref, eps_ref,
               wi_ref, bi_ref, wo_ref, bo_ref, wp1_ref, bp1_ref,
               wp2_ref, bp2_ref, xn_ref, pred_ref, hm_ref):
    i = pl.program_id(0)
    xn = _gin_update(x_ref, a0_ref, a1_ref, eps_ref, wi_ref, bi_ref,
                     wo_ref, bo_ref)
    xn_ref[...] = xn
    p = jnp.maximum(
        jnp.dot(xn, wp1_ref[...], preferred_element_type=jnp.float32)
        + bp1_ref[...], 0.0)
    pred_ref[...] = jnp.dot(p, wp2_ref[...],
                            preferred_element_type=jnp.float32) + bp2_ref[...]
    _pool_accum(i, nblk, xn, b_ref, cnt_ref, hm_ref)


def _full_spec(shape):
    return pl.BlockSpec(shape, lambda i: (0,) * len(shape))


def _row_spec():
    return pl.BlockSpec((RB, D), lambda i: (i, 0))


def _idx_spec():
    return pl.BlockSpec((1, 1, RB), lambda i: (i, 0, 0))


def kernel(d, index_vL, batch_vec, embed_d, layers, Wp1, bp1, Wp2, bp2):
    n = d.shape[0]
    e = index_vL.shape[1]
    nblk = (n + RB - 1) // RB
    npad = nblk * RB
    total_chunks = (e + CHUNK - 1) // CHUNK
    total_chunks += (-total_chunks) % (NSUB * 2 * PIECE)
    c0 = int(round(C0_FRAC * (total_chunks // NSUB) / PIECE)) * PIECE
    c0 = max(PIECE, min(total_chunks // NSUB - PIECE, c0))
    epad = CHUNK * total_chunks

    f32 = jnp.float32
    d_p = jnp.concatenate([d.astype(jnp.int32),
                           jnp.zeros((npad - n,), jnp.int32)])
    d3 = d_p.reshape(nblk, 1, RB)
    b_p = jnp.concatenate([batch_vec.astype(jnp.int32),
                           jnp.full((npad - n,), G, jnp.int32)])
    b3 = b_p.reshape(nblk, 1, RB)
    dst = index_vL[0].astype(jnp.int32)
    src = index_vL[1].astype(jnp.int32)
    src3 = jnp.concatenate([src, jnp.zeros((epad - e,), jnp.int32)]
                           ).reshape(total_chunks, CHUNK)
    dst3 = jnp.concatenate([dst, jnp.full((epad - e,), n, jnp.int32)]
                           ).reshape(total_chunks, CHUNK)
    emb_pad = jnp.zeros((8, D), f32).at[:MAX_DIS + 1].set(embed_d)
    zeros_hbm = jnp.zeros((npad // NSUB, D), f32)

    grid = (nblk,)
    row_out = jax.ShapeDtypeStruct((npad, D), f32)
    hm_out = jax.ShapeDtypeStruct((G, D), f32)

    x, m, counts = pl.pallas_call(
        _embed_body,
        grid=grid,
        in_specs=[_idx_spec(), _idx_spec(), _full_spec((8, D)),
                  _full_spec((D, D)), _full_spec((1, D))],
        out_specs=[_row_spec(), _row_spec(), _full_spec((G, D))],
        out_shape=[row_out, row_out, hm_out],
    )(d3, b3, emb_pad, layers[0]['Wl'], layers[0]['bl'].reshape(1, D))

    sc_agg = _make_sc_agg(npad, total_chunks, c0)

    mid_specs = [_row_spec(), _row_spec(), _row_spec(), _idx_spec(),
                 _full_spec((G, D)), _full_spec((1, 1)),
                 _full_spec((D, D)), _full_spec((1, D)),
                 _full_spec((D, D)), _full_spec((1, D))]

    h_means = []
    pred = None
    for li in range(len(layers)):
        lp = layers[li]
        parts = sc_agg(m, src3, dst3, zeros_hbm)
        a0, a1 = parts[0], parts[1]
        eps = lp['eps'].reshape(1, 1)
        if li + 1 < len(layers):
            nxt = layers[li + 1]
            x, m, hm = pl.pallas_call(
                functools.partial(_mid_body, nblk),
                grid=grid,
                in_specs=mid_specs + [_full_spec((D, D)), _full_spec((1, D))],
                out_specs=[_row_spec(), _row_spec(), _full_spec((G, D))],
                out_shape=[row_out, row_out, hm_out],
            )(x, a0, a1, b3, counts, eps,
              lp['Wi'], lp['bi'].reshape(1, D),
              lp['Wo'], lp['bo'].reshape(1, D),
              nxt['Wl'], nxt['bl'].reshape(1, D))
        else:
            wp2_pad = jnp.zeros((2 * D, D), f32).at[:, :1].set(Wp2)
            bp2_pad = jnp.zeros((1, D), f32).at[0, 0].set(bp2[0])
            x, pred, hm = pl.pallas_call(
                functools.partial(_last_body, nblk),
                grid=grid,
                in_specs=mid_specs + [_full_spec((D, 2 * D)),
                                      _full_spec((1, 2 * D)),
                                      _full_spec((2 * D, D)),
                                      _full_spec((1, D))],
                out_specs=[_row_spec(), _row_spec(), _full_spec((G, D))],
                out_shape=[row_out, row_out, hm_out],
            )(x, a0, a1, b3, counts, eps,
              lp['Wi'], lp['bi'].reshape(1, D),
              lp['Wo'], lp['bo'].reshape(1, D),
              Wp1, bp1.reshape(1, 2 * D), wp2_pad, bp2_pad)
        h_means.append(hm)

    return (pred[:n, :1], x[:n], tuple(h_means))


# speedup vs baseline: 1.1038x; 1.1038x over previous
import functools

import jax
import jax.numpy as jnp
from jax import lax
from jax.experimental import pallas as pl
from jax.experimental.pallas import tpu as pltpu
from jax.experimental.pallas import tpu_sc as plsc

D = 128
G = 64
MAX_DIS = 5
RB = 256
NW = 32
CHUNK = 128
NSUB = 16


NBUF = 2
PIECE = 40
C0_FRAC = 0.75


def _make_sc_agg(npad, total_chunks, c0):
    rows_per_tile = npad // NSUB
    c1 = total_chunks // NSUB - c0
    assert c0 % PIECE == 0 and c1 % PIECE == 0 and PIECE % NBUF == 0
    mesh = plsc.VectorSubcoreMesh(core_axis_name="c", subcore_axis_name="s")

    @functools.partial(
        pl.kernel,
        mesh=mesh,
        out_type=jax.ShapeDtypeStruct((2, npad, D), jnp.float32),
        scratch_types=[
            pltpu.VMEM((PIECE, CHUNK), jnp.int32),
            pltpu.VMEM((PIECE, CHUNK), jnp.int32),
            pltpu.VMEM((CHUNK, D), jnp.float32),
            pltpu.VMEM((CHUNK, D), jnp.float32),
            pltpu.VMEM_SHARED((npad, D), jnp.float32),
            pltpu.SemaphoreType.DMA,
            pltpu.SemaphoreType.DMA,
        ],
    )
    def sc_agg(m_hbm, src_hbm, dst_hbm, zeros_hbm, out_hbm,
               src_v, dst_v, b0, b1, agg, g0, g1):
        bufs = [b0, b1]
        gsems = [g0, g1]
        cid = lax.axis_index("c")
        sid = lax.axis_index("s")
        base = sid * rows_per_tile
        pltpu.sync_copy(zeros_hbm, agg.at[pl.ds(base, rows_per_tile)])
        plsc.subcore_barrier()

        mycount = c0 + cid * (c1 - c0)
        cbase = cid * NSUB * c0 + sid * mycount

        def piece_body(p, carry):
            off = cbase + p * PIECE
            pltpu.sync_copy(src_hbm.at[pl.ds(off, PIECE)], src_v)
            pltpu.sync_copy(dst_hbm.at[pl.ds(off, PIECE)], dst_v)

            for b in range(NBUF):
                pltpu.async_copy(m_hbm.at[src_v.at[b]], bufs[b], gsems[b])

            def body(i, carry2):
                j0 = i * NBUF
                for b in range(NBUF):
                    jn = j0 + b + NBUF
                    pltpu.make_async_copy(
                        m_hbm.at[src_v.at[j0 + b]], bufs[b], gsems[b]).wait()
                    pltpu.sync_copy(bufs[b], agg.at[dst_v.at[j0 + b]],
                                    add=True)

                    @pl.when(jn < PIECE)
                    def _():
                        pltpu.async_copy(m_hbm.at[src_v.at[jn]], bufs[b],
                                         gsems[b])
                return carry2

            lax.fori_loop(0, PIECE // NBUF, body, 0)
            return carry

        lax.fori_loop(0, mycount // PIECE, piece_body, 0)

        plsc.subcore_barrier()
        pltpu.sync_copy(agg.at[pl.ds(base, rows_per_tile)],
                        out_hbm.at[cid, pl.ds(base, rows_per_tile)])

    return sc_agg


def _embed_body(d_ref, b_ref, emb_ref, wl_ref, bl_ref,
                x_ref, m_ref, cnt_ref):
    i = pl.program_id(0)
    dd = jnp.clip(d_ref[0, 0, :], 0, MAX_DIS)
    oh = (lax.broadcasted_iota(jnp.int32, (RB, 8), 1) == dd[:, None])
    x = jnp.dot(oh.astype(jnp.float32), emb_ref[...],
                preferred_element_type=jnp.float32, precision=lax.Precision.HIGHEST)
    x_ref[...] = x
    m_ref[...] = jnp.maximum(
        jnp.dot(x, wl_ref[...], preferred_element_type=jnp.float32)
        + bl_ref[...], 0.0)
    bb = b_ref[0, 0, :]
    bsel = (lax.broadcasted_iota(jnp.int32, (G, RB), 0) == bb[None, :])
    cnt = jnp.dot(bsel.astype(jnp.float32), jnp.ones((RB, D), jnp.float32),
                  preferred_element_type=jnp.float32, precision=lax.Precision.HIGHEST)

    @pl.when(i == 0)
    def _():
        cnt_ref[...] = cnt

    @pl.when(i > 0)
    def _():
        cnt_ref[...] += cnt


def _gin_update(x_ref, a0_ref, a1_ref, eps_ref, wi_ref, bi_ref,
                wo_ref, bo_ref):
    h = x_ref[...] * (1.0 + eps_ref[0, 0]) + a0_ref[...] + a1_ref[...]
    t = jnp.maximum(
        jnp.dot(h, wi_ref[...], preferred_element_type=jnp.float32)
        + bi_ref[...], 0.0)
    h2 = jnp.dot(t, wo_ref[...], preferred_element_type=jnp.float32) \
        + bo_ref[...]
    return jnp.maximum(h2, 0.0)


def _pool_accum(i, nblk, xn, b_ref, cnt_ref, hm_ref):
    bb = b_ref[0, 0, :]
    bsel = (lax.broadcasted_iota(jnp.int32, (G, RB), 0) == bb[None, :])
    contrib = jnp.dot(bsel.astype(jnp.float32), xn,
                      preferred_element_type=jnp.float32, precision=lax.Precision.HIGHEST)

    @pl.when(i == 0)
    def _():
        hm_ref[...] = contrib

    @pl.when(i > 0)
    def _():
        hm_ref[...] += contrib

    @pl.when(i == nblk - 1)
    def _():
        hm_ref[...] = hm_ref[...] / jnp.maximum(cnt_ref[...], 1.0)


def _mid_body(nblk, x_ref, a0_ref, a1_ref, b_ref, cnt_ref, eps_ref,
              wi_ref, bi_ref, wo_ref, bo_ref, wn_ref, bn_ref,
              xn_ref, mn_ref, hm_ref):
    i = pl.program_id(0)
    xn = _gin_update(x_ref, a0_ref, a1_ref, eps_ref, wi_ref, bi_ref,
                     wo_ref, bo_ref)
    xn_ref[...] = xn
    mn_ref[...] = jnp.maximum(
        jnp.dot(xn, wn_ref[...], preferred_element_type=jnp.float32)
        + bn_ref[...], 0.0)
    _pool_accum(i, nblk, xn, b_ref, cnt_ref, hm_ref)


def _last_body(nblk, x_ref, a0_ref, a1_ref, b_ref, cnt_ref, eps_ref,
               wi_ref, bi_ref, wo_ref, bo_ref, wp1_ref, bp1_ref,
               wp2_ref, bp2_ref, xn_ref, pred_ref, hm_ref):
    i = pl.program_id(0)
    xn = _gin_update(x_ref, a0_ref, a1_ref, eps_ref, wi_ref, bi_ref,
                     wo_ref, bo_ref)
    xn_ref[...] = xn
    p = jnp.maximum(
        jnp.dot(xn, wp1_ref[...], preferred_element_type=jnp.float32)
        + bp1_ref[...], 0.0)
    pred_ref[...] = jnp.dot(p, wp2_ref[...],
                            preferred_element_type=jnp.float32) + bp2_ref[...]
    _pool_accum(i, nblk, xn, b_ref, cnt_ref, hm_ref)


def _full_spec(shape):
    return pl.BlockSpec(shape, lambda i: (0,) * len(shape))


def _row_spec():
    return pl.BlockSpec((RB, D), lambda i: (i, 0))


def _idx_spec():
    return pl.BlockSpec((1, 1, RB), lambda i: (i, 0, 0))


def kernel(d, index_vL, batch_vec, embed_d, layers, Wp1, bp1, Wp2, bp2):
    n = d.shape[0]
    e = index_vL.shape[1]
    nblk = (n + RB - 1) // RB
    npad = nblk * RB
    total_chunks = (e + CHUNK - 1) // CHUNK
    total_chunks += (-total_chunks) % (NSUB * 2 * PIECE)
    c0 = int(round(C0_FRAC * (total_chunks // NSUB) / PIECE)) * PIECE
    c0 = max(PIECE, min(total_chunks // NSUB - PIECE, c0))
    epad = CHUNK * total_chunks

    f32 = jnp.float32
    d_p = jnp.concatenate([d.astype(jnp.int32),
                           jnp.zeros((npad - n,), jnp.int32)])
    d3 = d_p.reshape(nblk, 1, RB)
    b_p = jnp.concatenate([batch_vec.astype(jnp.int32),
                           jnp.full((npad - n,), G, jnp.int32)])
    b3 = b_p.reshape(nblk, 1, RB)
    dst = index_vL[0].astype(jnp.int32)
    src = index_vL[1].astype(jnp.int32)
    src3 = jnp.concatenate([src, jnp.zeros((epad - e,), jnp.int32)]
                           ).reshape(total_chunks, CHUNK)
    dst3 = jnp.concatenate([dst, jnp.full((epad - e,), n, jnp.int32)]
                           ).reshape(total_chunks, CHUNK)
    emb_pad = jnp.zeros((8, D), f32).at[:MAX_DIS + 1].set(embed_d)
    zeros_hbm = jnp.zeros((npad // NSUB, D), f32)

    grid = (nblk,)
    row_out = jax.ShapeDtypeStruct((npad, D), f32)
    hm_out = jax.ShapeDtypeStruct((G, D), f32)

    x, m, counts = pl.pallas_call(
        _embed_body,
        grid=grid,
        in_specs=[_idx_spec(), _idx_spec(), _full_spec((8, D)),
                  _full_spec((D, D)), _full_spec((1, D))],
        out_specs=[_row_spec(), _row_spec(), _full_spec((G, D))],
        out_shape=[row_out, row_out, hm_out],
    )(d3, b3, emb_pad, layers[0]['Wl'], layers[0]['bl'].reshape(1, D))

    sc_agg = _make_sc_agg(npad, total_chunks, c0)

    mid_specs = [_row_spec(), _row_spec(), _row_spec(), _idx_spec(),
                 _full_spec((G, D)), _full_spec((1, 1)),
                 _full_spec((D, D)), _full_spec((1, D)),
                 _full_spec((D, D)), _full_spec((1, D))]

    h_means = []
    pred = None
    for li in range(len(layers)):
        lp = layers[li]
        parts = sc_agg(m, src3, dst3, zeros_hbm)
        a0, a1 = parts[0], parts[1]
        eps = lp['eps'].reshape(1, 1)
        if li + 1 < len(layers):
            nxt = layers[li + 1]
            x, m, hm = pl.pallas_call(
                functools.partial(_mid_body, nblk),
                grid=grid,
                in_specs=mid_specs + [_full_spec((D, D)), _full_spec((1, D))],
                out_specs=[_row_spec(), _row_spec(), _full_spec((G, D))],
                out_shape=[row_out, row_out, hm_out],
            )(x, a0, a1, b3, counts, eps,
              lp['Wi'], lp['bi'].reshape(1, D),
              lp['Wo'], lp['bo'].reshape(1, D),
              nxt['Wl'], nxt['bl'].reshape(1, D))
        else:
            wp2_pad = jnp.zeros((2 * D, D), f32).at[:, :1].set(Wp2)
            bp2_pad = jnp.zeros((1, D), f32).at[0, 0].set(bp2[0])
            x, pred, hm = pl.pallas_call(
                functools.partial(_last_body, nblk),
                grid=grid,
                in_specs=mid_specs + [_full_spec((D, 2 * D)),
                                      _full_spec((1, 2 * D)),
                                      _full_spec((2 * D, D)),
                                      _full_spec((1, D))],
                out_specs=[_row_spec(), _row_spec(), _full_spec((G, D))],
                out_shape=[row_out, row_out, hm_out],
            )(x, a0, a1, b3, counts, eps,
              lp['Wi'], lp['bi'].reshape(1, D),
              lp['Wo'], lp['bo'].reshape(1, D),
              Wp1, bp1.reshape(1, 2 * D), wp2_pad, bp2_pad)
        h_means.append(hm)

    return (pred[:n, :1], x[:n], tuple(h_means))


# speedup vs baseline: 1.2153x; 1.1010x over previous
import functools

import jax
import jax.numpy as jnp
from jax import lax
from jax.experimental import pallas as pl
from jax.experimental.pallas import tpu as pltpu
from jax.experimental.pallas import tpu_sc as plsc

D = 128
G = 64
MAX_DIS = 5
RB = 256
NW = 32
CHUNK = 128
NSUB = 16


NBUF = 2
PIECE = 8
C0_FRAC = 0.875


def _make_sc_agg(npad, total_chunks, c0):
    rows_per_tile = npad // NSUB
    c1 = total_chunks // NSUB - c0
    assert c0 % PIECE == 0 and c1 % PIECE == 0 and PIECE % NBUF == 0
    mesh = plsc.VectorSubcoreMesh(core_axis_name="c", subcore_axis_name="s")

    @functools.partial(
        pl.kernel,
        mesh=mesh,
        out_type=jax.ShapeDtypeStruct((2, npad, D), jnp.float32),
        scratch_types=[
            pltpu.VMEM((PIECE, CHUNK), jnp.int32),
            pltpu.VMEM((PIECE, CHUNK), jnp.int32),
            pltpu.VMEM((CHUNK, D), jnp.float32),
            pltpu.VMEM((CHUNK, D), jnp.float32),
            pltpu.VMEM_SHARED((npad, D), jnp.float32),
            pltpu.SemaphoreType.DMA,
            pltpu.SemaphoreType.DMA,
        ],
    )
    def sc_agg(m_hbm, src_hbm, dst_hbm, zeros_hbm, out_hbm,
               src_v, dst_v, b0, b1, agg, g0, g1):
        bufs = [b0, b1]
        gsems = [g0, g1]
        cid = lax.axis_index("c")
        sid = lax.axis_index("s")
        base = sid * rows_per_tile
        pltpu.sync_copy(zeros_hbm, agg.at[pl.ds(base, rows_per_tile)])
        plsc.subcore_barrier()

        mycount = c0 + cid * (c1 - c0)
        cbase = cid * NSUB * c0 + sid * mycount

        def piece_body(p, carry):
            off = cbase + p * PIECE
            pltpu.sync_copy(src_hbm.at[pl.ds(off, PIECE)], src_v)
            pltpu.sync_copy(dst_hbm.at[pl.ds(off, PIECE)], dst_v)

            for b in range(NBUF):
                pltpu.async_copy(m_hbm.at[src_v.at[b]], bufs[b], gsems[b])

            def body(i, carry2):
                j0 = i * NBUF
                for b in range(NBUF):
                    jn = j0 + b + NBUF
                    pltpu.make_async_copy(
                        m_hbm.at[src_v.at[j0 + b]], bufs[b], gsems[b]).wait()
                    pltpu.sync_copy(bufs[b], agg.at[dst_v.at[j0 + b]],
                                    add=True)

                    @pl.when(jn < PIECE)
                    def _():
                        pltpu.async_copy(m_hbm.at[src_v.at[jn]], bufs[b],
                                         gsems[b])
                return carry2

            lax.fori_loop(0, PIECE // NBUF, body, 0)
            return carry

        lax.fori_loop(0, mycount // PIECE, piece_body, 0)

        plsc.subcore_barrier()
        pltpu.sync_copy(agg.at[pl.ds(base, rows_per_tile)],
                        out_hbm.at[cid, pl.ds(base, rows_per_tile)])

    return sc_agg


def _embed_body(d_ref, b_ref, emb_ref, wl_ref, bl_ref,
                x_ref, m_ref, cnt_ref):
    i = pl.program_id(0)
    dd = jnp.clip(d_ref[0, 0, :], 0, MAX_DIS)
    oh = (lax.broadcasted_iota(jnp.int32, (RB, 8), 1) == dd[:, None])
    x = jnp.dot(oh.astype(jnp.float32), emb_ref[...],
                preferred_element_type=jnp.float32, precision=lax.Precision.HIGHEST)
    x_ref[...] = x
    m_ref[...] = jnp.maximum(
        jnp.dot(x, wl_ref[...], preferred_element_type=jnp.float32)
        + bl_ref[...], 0.0)
    bb = b_ref[0, 0, :]
    bsel = (lax.broadcasted_iota(jnp.int32, (G, RB), 0) == bb[None, :])
    cnt = jnp.dot(bsel.astype(jnp.float32), jnp.ones((RB, D), jnp.float32),
                  preferred_element_type=jnp.float32, precision=lax.Precision.HIGHEST)

    @pl.when(i == 0)
    def _():
        cnt_ref[...] = cnt

    @pl.when(i > 0)
    def _():
        cnt_ref[...] += cnt


def _gin_update(x_ref, a0_ref, a1_ref, eps_ref, wi_ref, bi_ref,
                wo_ref, bo_ref):
    h = x_ref[...] * (1.0 + eps_ref[0, 0]) + a0_ref[...] + a1_ref[...]
    t = jnp.maximum(
        jnp.dot(h, wi_ref[...], preferred_element_type=jnp.float32)
        + bi_ref[...], 0.0)
    h2 = jnp.dot(t, wo_ref[...], preferred_element_type=jnp.float32) \
        + bo_ref[...]
    return jnp.maximum(h2, 0.0)


def _pool_accum(i, nblk, xn, b_ref, cnt_ref, hm_ref):
    bb = b_ref[0, 0, :]
    bsel = (lax.broadcasted_iota(jnp.int32, (G, RB), 0) == bb[None, :])
    contrib = jnp.dot(bsel.astype(jnp.float32), xn,
                      preferred_element_type=jnp.float32, precision=lax.Precision.HIGHEST)

    @pl.when(i == 0)
    def _():
        hm_ref[...] = contrib

    @pl.when(i > 0)
    def _():
        hm_ref[...] += contrib

    @pl.when(i == nblk - 1)
    def _():
        hm_ref[...] = hm_ref[...] / jnp.maximum(cnt_ref[...], 1.0)


def _mid_body(nblk, x_ref, a0_ref, a1_ref, b_ref, cnt_ref, eps_ref,
              wi_ref, bi_ref, wo_ref, bo_ref, wn_ref, bn_ref,
              xn_ref, mn_ref, hm_ref):
    i = pl.program_id(0)
    xn = _gin_update(x_ref, a0_ref, a1_ref, eps_ref, wi_ref, bi_ref,
                     wo_ref, bo_ref)
    xn_ref[...] = xn
    mn_ref[...] = jnp.maximum(
        jnp.dot(xn, wn_ref[...], preferred_element_type=jnp.float32)
        + bn_ref[...], 0.0)
    _pool_accum(i, nblk, xn, b_ref, cnt_ref, hm_ref)


def _last_body(nblk, x_ref, a0_ref, a1_ref, b_ref, cnt_ref, eps_ref,
               wi_ref, bi_ref, wo_ref, bo_ref, wp1_ref, bp1_ref,
               wp2_ref, bp2_ref, xn_ref, pred_ref, hm_ref):
    i = pl.program_id(0)
    xn = _gin_update(x_ref, a0_ref, a1_ref, eps_ref, wi_ref, bi_ref,
                     wo_ref, bo_ref)
    xn_ref[...] = xn
    p = jnp.maximum(
        jnp.dot(xn, wp1_ref[...], preferred_element_type=jnp.float32)
        + bp1_ref[...], 0.0)
    pred_ref[...] = jnp.dot(p, wp2_ref[...],
                            preferred_element_type=jnp.float32) + bp2_ref[...]
    _pool_accum(i, nblk, xn, b_ref, cnt_ref, hm_ref)


def _full_spec(shape):
    return pl.BlockSpec(shape, lambda i: (0,) * len(shape))


def _row_spec():
    return pl.BlockSpec((RB, D), lambda i: (i, 0))


def _idx_spec():
    return pl.BlockSpec((1, 1, RB), lambda i: (i, 0, 0))


def kernel(d, index_vL, batch_vec, embed_d, layers, Wp1, bp1, Wp2, bp2):
    n = d.shape[0]
    e = index_vL.shape[1]
    nblk = (n + RB - 1) // RB
    npad = nblk * RB
    total_chunks = (e + CHUNK - 1) // CHUNK
    total_chunks += (-total_chunks) % (NSUB * 2 * PIECE)
    c0 = int(round(C0_FRAC * (total_chunks // NSUB) / PIECE)) * PIECE
    c0 = max(PIECE, min(total_chunks // NSUB - PIECE, c0))
    epad = CHUNK * total_chunks

    f32 = jnp.float32
    d_p = jnp.concatenate([d.astype(jnp.int32),
                           jnp.zeros((npad - n,), jnp.int32)])
    d3 = d_p.reshape(nblk, 1, RB)
    b_p = jnp.concatenate([batch_vec.astype(jnp.int32),
                           jnp.full((npad - n,), G, jnp.int32)])
    b3 = b_p.reshape(nblk, 1, RB)
    dst = index_vL[0].astype(jnp.int32)
    src = index_vL[1].astype(jnp.int32)
    src3 = jnp.concatenate([src, jnp.zeros((epad - e,), jnp.int32)]
                           ).reshape(total_chunks, CHUNK)
    dst3 = jnp.concatenate([dst, jnp.full((epad - e,), n, jnp.int32)]
                           ).reshape(total_chunks, CHUNK)
    emb_pad = jnp.zeros((8, D), f32).at[:MAX_DIS + 1].set(embed_d)
    zeros_hbm = jnp.zeros((npad // NSUB, D), f32)

    grid = (nblk,)
    row_out = jax.ShapeDtypeStruct((npad, D), f32)
    hm_out = jax.ShapeDtypeStruct((G, D), f32)

    x, m, counts = pl.pallas_call(
        _embed_body,
        grid=grid,
        in_specs=[_idx_spec(), _idx_spec(), _full_spec((8, D)),
                  _full_spec((D, D)), _full_spec((1, D))],
        out_specs=[_row_spec(), _row_spec(), _full_spec((G, D))],
        out_shape=[row_out, row_out, hm_out],
    )(d3, b3, emb_pad, layers[0]['Wl'], layers[0]['bl'].reshape(1, D))

    sc_agg = _make_sc_agg(npad, total_chunks, c0)

    mid_specs = [_row_spec(), _row_spec(), _row_spec(), _idx_spec(),
                 _full_spec((G, D)), _full_spec((1, 1)),
                 _full_spec((D, D)), _full_spec((1, D)),
                 _full_spec((D, D)), _full_spec((1, D))]

    h_means = []
    pred = None
    for li in range(len(layers)):
        lp = layers[li]
        parts = sc_agg(m, src3, dst3, zeros_hbm)
        a0, a1 = parts[0], parts[1]
        eps = lp['eps'].reshape(1, 1)
        if li + 1 < len(layers):
            nxt = layers[li + 1]
            x, m, hm = pl.pallas_call(
                functools.partial(_mid_body, nblk),
                grid=grid,
                in_specs=mid_specs + [_full_spec((D, D)), _full_spec((1, D))],
                out_specs=[_row_spec(), _row_spec(), _full_spec((G, D))],
                out_shape=[row_out, row_out, hm_out],
            )(x, a0, a1, b3, counts, eps,
              lp['Wi'], lp['bi'].reshape(1, D),
              lp['Wo'], lp['bo'].reshape(1, D),
              nxt['Wl'], nxt['bl'].reshape(1, D))
        else:
            wp2_pad = jnp.zeros((2 * D, D), f32).at[:, :1].set(Wp2)
            bp2_pad = jnp.zeros((1, D), f32).at[0, 0].set(bp2[0])
            x, pred, hm = pl.pallas_call(
                functools.partial(_last_body, nblk),
                grid=grid,
                in_specs=mid_specs + [_full_spec((D, 2 * D)),
                                      _full_spec((1, 2 * D)),
                                      _full_spec((2 * D, D)),
                                      _full_spec((1, D))],
                out_specs=[_row_spec(), _row_spec(), _full_spec((G, D))],
                out_shape=[row_out, row_out, hm_out],
            )(x, a0, a1, b3, counts, eps,
              lp['Wi'], lp['bi'].reshape(1, D),
              lp['Wo'], lp['bo'].reshape(1, D),
              Wp1, bp1.reshape(1, 2 * D), wp2_pad, bp2_pad)
        h_means.append(hm)

    return (pred[:n, :1], x[:n], tuple(h_means))


# speedup vs baseline: 1.2251x; 1.0081x over previous
import functools

import jax
import jax.numpy as jnp
from jax import lax
from jax.experimental import pallas as pl
from jax.experimental.pallas import tpu as pltpu
from jax.experimental.pallas import tpu_sc as plsc

D = 128
G = 64
MAX_DIS = 5
RB = 256
NW = 32
CHUNK = 128
NSUB = 16


NBUF = 2
PIECE = 8
C0_FRAC = 0.95


def _make_sc_agg(npad, total_chunks, c0):
    rows_per_tile = npad // NSUB
    c1 = total_chunks // NSUB - c0
    assert c0 % PIECE == 0 and c1 % PIECE == 0 and PIECE % NBUF == 0
    mesh = plsc.VectorSubcoreMesh(core_axis_name="c", subcore_axis_name="s")

    @functools.partial(
        pl.kernel,
        mesh=mesh,
        out_type=jax.ShapeDtypeStruct((2, npad, D), jnp.float32),
        scratch_types=[
            pltpu.VMEM((PIECE, CHUNK), jnp.int32),
            pltpu.VMEM((PIECE, CHUNK), jnp.int32),
            pltpu.VMEM((CHUNK, D), jnp.float32),
            pltpu.VMEM((CHUNK, D), jnp.float32),
            pltpu.VMEM_SHARED((npad, D), jnp.float32),
            pltpu.SemaphoreType.DMA,
            pltpu.SemaphoreType.DMA,
        ],
    )
    def sc_agg(m_hbm, src_hbm, dst_hbm, zeros_hbm, out_hbm,
               src_v, dst_v, b0, b1, agg, g0, g1):
        bufs = [b0, b1]
        gsems = [g0, g1]
        cid = lax.axis_index("c")
        sid = lax.axis_index("s")
        base = sid * rows_per_tile
        pltpu.sync_copy(zeros_hbm, agg.at[pl.ds(base, rows_per_tile)])
        plsc.subcore_barrier()

        mycount = c0 + cid * (c1 - c0)
        cbase = cid * NSUB * c0 + sid * mycount

        def piece_body(p, carry):
            off = cbase + p * PIECE
            pltpu.sync_copy(src_hbm.at[pl.ds(off, PIECE)], src_v)
            pltpu.sync_copy(dst_hbm.at[pl.ds(off, PIECE)], dst_v)

            for b in range(NBUF):
                pltpu.async_copy(m_hbm.at[src_v.at[b]], bufs[b], gsems[b])

            def body(i, carry2):
                j0 = i * NBUF
                for b in range(NBUF):
                    jn = j0 + b + NBUF
                    pltpu.make_async_copy(
                        m_hbm.at[src_v.at[j0 + b]], bufs[b], gsems[b]).wait()
                    pltpu.sync_copy(bufs[b], agg.at[dst_v.at[j0 + b]],
                                    add=True)

                    @pl.when(jn < PIECE)
                    def _():
                        pltpu.async_copy(m_hbm.at[src_v.at[jn]], bufs[b],
                                         gsems[b])
                return carry2

            lax.fori_loop(0, PIECE // NBUF, body, 0)
            return carry

        lax.fori_loop(0, mycount // PIECE, piece_body, 0)

        plsc.subcore_barrier()
        pltpu.sync_copy(agg.at[pl.ds(base, rows_per_tile)],
                        out_hbm.at[cid, pl.ds(base, rows_per_tile)])

    return sc_agg


def _embed_body(d_ref, b_ref, emb_ref, wl_ref, bl_ref,
                x_ref, m_ref, cnt_ref):
    i = pl.program_id(0)
    dd = jnp.clip(d_ref[0, 0, :], 0, MAX_DIS)
    oh = (lax.broadcasted_iota(jnp.int32, (RB, 8), 1) == dd[:, None])
    x = jnp.dot(oh.astype(jnp.float32), emb_ref[...],
                preferred_element_type=jnp.float32, precision=lax.Precision.HIGHEST)
    x_ref[...] = x
    m_ref[...] = jnp.maximum(
        jnp.dot(x, wl_ref[...], preferred_element_type=jnp.float32)
        + bl_ref[...], 0.0)
    bb = b_ref[0, 0, :]
    bsel = (lax.broadcasted_iota(jnp.int32, (G, RB), 0) == bb[None, :])
    cnt = jnp.dot(bsel.astype(jnp.float32), jnp.ones((RB, D), jnp.float32),
                  preferred_element_type=jnp.float32, precision=lax.Precision.HIGHEST)

    @pl.when(i == 0)
    def _():
        cnt_ref[...] = cnt

    @pl.when(i > 0)
    def _():
        cnt_ref[...] += cnt


def _gin_update(x_ref, a0_ref, a1_ref, eps_ref, wi_ref, bi_ref,
                wo_ref, bo_ref):
    h = x_ref[...] * (1.0 + eps_ref[0, 0]) + a0_ref[...] + a1_ref[...]
    t = jnp.maximum(
        jnp.dot(h, wi_ref[...], preferred_element_type=jnp.float32)
        + bi_ref[...], 0.0)
    h2 = jnp.dot(t, wo_ref[...], preferred_element_type=jnp.float32) \
        + bo_ref[...]
    return jnp.maximum(h2, 0.0)


def _pool_accum(i, nblk, xn, b_ref, cnt_ref, hm_ref):
    bb = b_ref[0, 0, :]
    bsel = (lax.broadcasted_iota(jnp.int32, (G, RB), 0) == bb[None, :])
    contrib = jnp.dot(bsel.astype(jnp.float32), xn,
                      preferred_element_type=jnp.float32, precision=lax.Precision.HIGHEST)

    @pl.when(i == 0)
    def _():
        hm_ref[...] = contrib

    @pl.when(i > 0)
    def _():
        hm_ref[...] += contrib

    @pl.when(i == nblk - 1)
    def _():
        hm_ref[...] = hm_ref[...] / jnp.maximum(cnt_ref[...], 1.0)


def _mid_body(nblk, x_ref, a0_ref, a1_ref, b_ref, cnt_ref, eps_ref,
              wi_ref, bi_ref, wo_ref, bo_ref, wn_ref, bn_ref,
              xn_ref, mn_ref, hm_ref):
    i = pl.program_id(0)
    xn = _gin_update(x_ref, a0_ref, a1_ref, eps_ref, wi_ref, bi_ref,
                     wo_ref, bo_ref)
    xn_ref[...] = xn
    mn_ref[...] = jnp.maximum(
        jnp.dot(xn, wn_ref[...], preferred_element_type=jnp.float32)
        + bn_ref[...], 0.0)
    _pool_accum(i, nblk, xn, b_ref, cnt_ref, hm_ref)


def _last_body(nblk, x_ref, a0_ref, a1_ref, b_ref, cnt_ref, eps_ref,
               wi_ref, bi_ref, wo_ref, bo_ref, wp1_ref, bp1_ref,
               wp2_ref, bp2_ref, xn_ref, pred_ref, hm_ref):
    i = pl.program_id(0)
    xn = _gin_update(x_ref, a0_ref, a1_ref, eps_ref, wi_ref, bi_ref,
                     wo_ref, bo_ref)
    xn_ref[...] = xn
    p = jnp.maximum(
        jnp.dot(xn, wp1_ref[...], preferred_element_type=jnp.float32)
        + bp1_ref[...], 0.0)
    pred_ref[...] = jnp.dot(p, wp2_ref[...],
                            preferred_element_type=jnp.float32) + bp2_ref[...]
    _pool_accum(i, nblk, xn, b_ref, cnt_ref, hm_ref)


def _full_spec(shape):
    return pl.BlockSpec(shape, lambda i: (0,) * len(shape))


def _row_spec():
    return pl.BlockSpec((RB, D), lambda i: (i, 0))


def _idx_spec():
    return pl.BlockSpec((1, 1, RB), lambda i: (i, 0, 0))


def kernel(d, index_vL, batch_vec, embed_d, layers, Wp1, bp1, Wp2, bp2):
    n = d.shape[0]
    e = index_vL.shape[1]
    nblk = (n + RB - 1) // RB
    npad = nblk * RB
    total_chunks = (e + CHUNK - 1) // CHUNK
    total_chunks += (-total_chunks) % (NSUB * 2 * PIECE)
    c0 = int(round(C0_FRAC * (total_chunks // NSUB) / PIECE)) * PIECE
    c0 = max(PIECE, min(total_chunks // NSUB - PIECE, c0))
    epad = CHUNK * total_chunks

    f32 = jnp.float32
    d_p = jnp.concatenate([d.astype(jnp.int32),
                           jnp.zeros((npad - n,), jnp.int32)])
    d3 = d_p.reshape(nblk, 1, RB)
    b_p = jnp.concatenate([batch_vec.astype(jnp.int32),
                           jnp.full((npad - n,), G, jnp.int32)])
    b3 = b_p.reshape(nblk, 1, RB)
    dst = index_vL[0].astype(jnp.int32)
    src = index_vL[1].astype(jnp.int32)
    src3 = jnp.concatenate([src, jnp.zeros((epad - e,), jnp.int32)]
                           ).reshape(total_chunks, CHUNK)
    dst3 = jnp.concatenate([dst, jnp.full((epad - e,), n, jnp.int32)]
                           ).reshape(total_chunks, CHUNK)
    emb_pad = jnp.zeros((8, D), f32).at[:MAX_DIS + 1].set(embed_d)
    zeros_hbm = jnp.zeros((npad // NSUB, D), f32)

    grid = (nblk,)
    row_out = jax.ShapeDtypeStruct((npad, D), f32)
    hm_out = jax.ShapeDtypeStruct((G, D), f32)

    x, m, counts = pl.pallas_call(
        _embed_body,
        grid=grid,
        in_specs=[_idx_spec(), _idx_spec(), _full_spec((8, D)),
                  _full_spec((D, D)), _full_spec((1, D))],
        out_specs=[_row_spec(), _row_spec(), _full_spec((G, D))],
        out_shape=[row_out, row_out, hm_out],
    )(d3, b3, emb_pad, layers[0]['Wl'], layers[0]['bl'].reshape(1, D))

    sc_agg = _make_sc_agg(npad, total_chunks, c0)

    mid_specs = [_row_spec(), _row_spec(), _row_spec(), _idx_spec(),
                 _full_spec((G, D)), _full_spec((1, 1)),
                 _full_spec((D, D)), _full_spec((1, D)),
                 _full_spec((D, D)), _full_spec((1, D))]

    h_means = []
    pred = None
    for li in range(len(layers)):
        lp = layers[li]
        parts = sc_agg(m, src3, dst3, zeros_hbm)
        a0, a1 = parts[0], parts[1]
        eps = lp['eps'].reshape(1, 1)
        if li + 1 < len(layers):
            nxt = layers[li + 1]
            x, m, hm = pl.pallas_call(
                functools.partial(_mid_body, nblk),
                grid=grid,
                in_specs=mid_specs + [_full_spec((D, D)), _full_spec((1, D))],
                out_specs=[_row_spec(), _row_spec(), _full_spec((G, D))],
                out_shape=[row_out, row_out, hm_out],
            )(x, a0, a1, b3, counts, eps,
              lp['Wi'], lp['bi'].reshape(1, D),
              lp['Wo'], lp['bo'].reshape(1, D),
              nxt['Wl'], nxt['bl'].reshape(1, D))
        else:
            wp2_pad = jnp.zeros((2 * D, D), f32).at[:, :1].set(Wp2)
            bp2_pad = jnp.zeros((1, D), f32).at[0, 0].set(bp2[0])
            x, pred, hm = pl.pallas_call(
                functools.partial(_last_body, nblk),
                grid=grid,
                in_specs=mid_specs + [_full_spec((D, 2 * D)),
                                      _full_spec((1, 2 * D)),
                                      _full_spec((2 * D, D)),
                                      _full_spec((1, D))],
                out_specs=[_row_spec(), _row_spec(), _full_spec((G, D))],
                out_shape=[row_out, row_out, hm_out],
            )(x, a0, a1, b3, counts, eps,
              lp['Wi'], lp['bi'].reshape(1, D),
              lp['Wo'], lp['bo'].reshape(1, D),
              Wp1, bp1.reshape(1, 2 * D), wp2_pad, bp2_pad)
        h_means.append(hm)

    return (pred[:n, :1], x[:n], tuple(h_means))
